# baseline (speedup 1.0000x reference)
import functools

import jax
import jax.numpy as jnp
from jax import lax
from jax.experimental import pallas as pl
from jax.experimental.pallas import tpu as pltpu
from jax.experimental.pallas import tpu_sc as plsc

N = 10000
D = 128
E = 320000
NC = 2
NS = 16
NW = NC * NS
K = 128
CH = 80
SCH = 40
EPW = CH * K
EPAD = NW * EPW
NP = 10112
RPZ = NP // NS
DW = D

_MESH = plsc.VectorSubcoreMesh(core_axis_name="c", subcore_axis_name="s")


@functools.partial(
    pl.kernel,
    out_type=jax.ShapeDtypeStruct((NC, NP, D), jnp.float32),
    mesh=_MESH,
    scratch_types=[
        pltpu.VMEM((CH, K), jnp.int32),
        pltpu.VMEM((K, D), jnp.float32),
        pltpu.VMEM_SHARED((NP, D), jnp.float32),
    ],
)
def _deg_kernel(dsts_hbm, ones_hbm, zeros_hbm, out_hbm, dst_v, ones_v, dacc):
    c = lax.axis_index("c")
    s = lax.axis_index("s")
    wid = c * NS + s
    pltpu.sync_copy(zeros_hbm.at[pl.ds(s * RPZ, RPZ)], dacc.at[pl.ds(s * RPZ, RPZ)])
    pltpu.sync_copy(dsts_hbm.at[wid], dst_v)
    pltpu.sync_copy(ones_hbm, ones_v)
    plsc.subcore_barrier()

    def body(j, carry):
        pltpu.sync_copy(ones_v, dacc.at[dst_v.at[j]], add=True)
        return carry

    lax.fori_loop(0, CH, body, 0)
    plsc.subcore_barrier()
    pltpu.sync_copy(dacc.at[pl.ds(s * RPZ, RPZ)], out_hbm.at[c, pl.ds(s * RPZ, RPZ)])


@functools.partial(
    pl.kernel,
    out_type=jax.ShapeDtypeStruct((NC, NP, D), jnp.float32),
    mesh=_MESH,
    scratch_types=[
        pltpu.VMEM((SCH, K), jnp.int32),
        pltpu.VMEM((SCH, K), jnp.int32),
        pltpu.VMEM((K, D), jnp.float32),
        pltpu.VMEM((K, D), jnp.float32),
        pltpu.VMEM_SHARED((NP, D), jnp.float32),
        pltpu.SemaphoreType.DMA,
        pltpu.SemaphoreType.DMA,
    ],
)
def _agg_kernel(hp_hbm, srcs_hbm, dsts_hbm, zeros_hbm, out_hbm,
                src_v, dst_v, rows0, rows1, acc, sem0, sem1):
    c = lax.axis_index("c")
    s = lax.axis_index("s")
    wid = c * NS + s
    pltpu.sync_copy(zeros_hbm.at[pl.ds(s * RPZ, RPZ)], acc.at[pl.ds(s * RPZ, RPZ)])

    for h in range(CH // SCH):
        pltpu.sync_copy(srcs_hbm.at[wid, pl.ds(h * SCH, SCH)], src_v)
        pltpu.sync_copy(dsts_hbm.at[wid, pl.ds(h * SCH, SCH)], dst_v)
        if h == 0:
            plsc.subcore_barrier()
        pltpu.async_copy(hp_hbm.at[src_v.at[0]], rows0, sem0)

        def body(i, carry):
            j0 = 2 * i
            j1 = j0 + 1
            pltpu.async_copy(hp_hbm.at[src_v.at[j1]], rows1, sem1)
            pltpu.make_async_copy(hp_hbm.at[src_v.at[j0]], rows0, sem0).wait()
            pltpu.sync_copy(rows0, acc.at[dst_v.at[j0]], add=True)

            @pl.when(j0 + 2 < SCH)
            def _():
                pltpu.async_copy(hp_hbm.at[src_v.at[j0 + 2]], rows0, sem0)

            pltpu.make_async_copy(hp_hbm.at[src_v.at[j1]], rows1, sem1).wait()
            pltpu.sync_copy(rows1, acc.at[dst_v.at[j1]], add=True)
            return carry

        lax.fori_loop(0, SCH // 2, body, 0)
    plsc.subcore_barrier()
    pltpu.sync_copy(acc.at[pl.ds(s * RPZ, RPZ)], out_hbm.at[c, pl.ds(s * RPZ, RPZ)])


_RB = 400
_GRID = N // _RB

_row = pl.BlockSpec((_RB, D), lambda i: (i, 0))
_rowcol = pl.BlockSpec((_RB, 1), lambda i: (i, 0))
_wspec = pl.BlockSpec((D, D), lambda i: (0, 0))
_bspec = pl.BlockSpec((1, D), lambda i: (0, 0))
_degspec = pl.BlockSpec((NC, _RB, DW), lambda i: (0, i, 0))
_p0spec = pl.BlockSpec((1, _RB, D), lambda i: (0, i, 0))
_p1spec = pl.BlockSpec((1, _RB, D), lambda i: (1, i, 0))


def _tc1_body(degp_ref, x_ref, w_ref, hp_ref, dinv_ref):
    deg = degp_ref[0, :, :1] + degp_ref[1, :, :1] + 1.0
    dinv = lax.rsqrt(deg)
    dinv_ref[...] = dinv
    hp_ref[...] = jnp.dot(x_ref[...], w_ref[...],
                          preferred_element_type=jnp.float32) * dinv


_tc1 = pl.pallas_call(
    _tc1_body,
    grid=(_GRID,),
    in_specs=[_degspec, _row, _wspec],
    out_specs=(_row, _rowcol),
    out_shape=(jax.ShapeDtypeStruct((N, D), jnp.float32),
               jax.ShapeDtypeStruct((N, 1), jnp.float32)),
)


def _tc_mid_body(hp_ref, a0_ref, a1_ref, dinv_ref, b_ref, w_ref, out_ref):
    dinv = dinv_ref[...]
    z = dinv * (hp_ref[...] + a0_ref[0] + a1_ref[0]) + b_ref[...]
    h = jnp.maximum(z, 0.0)
    out_ref[...] = jnp.dot(h, w_ref[...],
                           preferred_element_type=jnp.float32) * dinv


_tc_mid = pl.pallas_call(
    _tc_mid_body,
    grid=(_GRID,),
    in_specs=[_row, _p0spec, _p1spec, _rowcol, _bspec, _wspec],
    out_specs=_row,
    out_shape=jax.ShapeDtypeStruct((N, D), jnp.float32),
)


def _tc_final_body(hp_ref, a0_ref, a1_ref, dinv_ref, b_ref, out_ref):
    out_ref[...] = (dinv_ref[...] * (hp_ref[...] + a0_ref[0] + a1_ref[0])
                    + b_ref[...])


_tc_final = pl.pallas_call(
    _tc_final_body,
    grid=(_GRID,),
    in_specs=[_row, _p0spec, _p1spec, _rowcol, _bspec],
    out_specs=_row,
    out_shape=jax.ShapeDtypeStruct((N, D), jnp.float32),
)


def kernel(x, edge_index, W1, b1, W2, b2, W3, b3):
    ei = edge_index.astype(jnp.int32)
    pad = EPAD - E
    src = jnp.concatenate([ei[0], jnp.zeros((pad,), jnp.int32)]).reshape(NW, CH, K)
    dst = jnp.concatenate([ei[1], jnp.full((pad,), N, jnp.int32)]).reshape(NW, CH, K)

    onesD = jnp.ones((K, D), jnp.float32)
    zerosD = jnp.zeros((NP, D), jnp.float32)

    degp = _deg_kernel(dst, onesD, zerosD)
    h1p, dinv = _tc1(degp, x, W1)

    b1r = b1.reshape(1, D)
    b2r = b2.reshape(1, D)
    b3r = b3.reshape(1, D)

    a1 = _agg_kernel(h1p, src, dst, zerosD)
    h2p = _tc_mid(h1p, a1, a1, dinv, b1r, W2)
    a2 = _agg_kernel(h2p, src, dst, zerosD)
    h3p = _tc_mid(h2p, a2, a2, dinv, b2r, W3)
    a3 = _agg_kernel(h3p, src, dst, zerosD)
    out = _tc_final(h3p, a3, a3, dinv, b3r)
    return out


# speedup vs baseline: 1.1824x; 1.1824x over previous
import functools

import jax
import jax.numpy as jnp
from jax import lax
from jax.experimental import pallas as pl
from jax.experimental.pallas import tpu as pltpu
from jax.experimental.pallas import tpu_sc as plsc

N = 10000
D = 128
E = 320000
NC = 2
NS = 16
NW = NC * NS
K = 128
CH = 80
SCH = 40
EPW = CH * K
EPAD = NW * EPW
NP = 10112
RPZ = NP // NS
DW = D

_MESH = plsc.VectorSubcoreMesh(core_axis_name="c", subcore_axis_name="s")


@functools.partial(
    pl.kernel,
    out_type=jax.ShapeDtypeStruct((NC, NP, D), jnp.float32),
    mesh=_MESH,
    scratch_types=[
        pltpu.VMEM((CH, K), jnp.int32),
        pltpu.VMEM((K, D), jnp.float32),
        pltpu.VMEM_SHARED((NP, D), jnp.float32),
    ],
)
def _deg_kernel(dsts_hbm, ones_hbm, zeros_hbm, out_hbm, dst_v, ones_v, dacc):
    c = lax.axis_index("c")
    s = lax.axis_index("s")
    wid = c * NS + s
    pltpu.sync_copy(zeros_hbm.at[pl.ds(s * RPZ, RPZ)], dacc.at[pl.ds(s * RPZ, RPZ)])
    pltpu.sync_copy(dsts_hbm.at[wid], dst_v)
    pltpu.sync_copy(ones_hbm, ones_v)
    plsc.subcore_barrier()

    def body(j, carry):
        pltpu.sync_copy(ones_v, dacc.at[dst_v.at[j]], add=True)
        return carry

    lax.fori_loop(0, CH, body, 0)
    plsc.subcore_barrier()
    pltpu.sync_copy(dacc.at[pl.ds(s * RPZ, RPZ)], out_hbm.at[c, pl.ds(s * RPZ, RPZ)])


@functools.partial(
    pl.kernel,
    out_type=jax.ShapeDtypeStruct((NC, NP, D), jnp.float32),
    mesh=_MESH,
    scratch_types=[
        pltpu.VMEM((SCH, K), jnp.int32),
        pltpu.VMEM((SCH, K), jnp.int32),
        pltpu.VMEM((K, D), jnp.float32),
        pltpu.VMEM((K, D), jnp.float32),
        pltpu.VMEM_SHARED((NP, D), jnp.float32),
        pltpu.SemaphoreType.DMA,
        pltpu.SemaphoreType.DMA,
    ],
)
def _agg_kernel(hp_hbm, srcs_hbm, dsts_hbm, zeros_hbm, out_hbm,
                src_v, dst_v, rows0, rows1, acc, sem0, sem1):
    c = lax.axis_index("c")
    s = lax.axis_index("s")
    wid = c * NS + s
    pltpu.sync_copy(zeros_hbm.at[pl.ds(s * RPZ, RPZ)], acc.at[pl.ds(s * RPZ, RPZ)])

    for h in range(CH // SCH):
        pltpu.sync_copy(srcs_hbm.at[wid, pl.ds(h * SCH, SCH)], src_v)
        pltpu.sync_copy(dsts_hbm.at[wid, pl.ds(h * SCH, SCH)], dst_v)
        if h == 0:
            plsc.subcore_barrier()
        pltpu.async_copy(hp_hbm.at[src_v.at[0]], rows0, sem0)

        def body(i, carry):
            j0 = 2 * i
            j1 = j0 + 1
            pltpu.async_copy(hp_hbm.at[src_v.at[j1]], rows1, sem1)
            pltpu.make_async_copy(hp_hbm.at[src_v.at[j0]], rows0, sem0).wait()
            pltpu.sync_copy(rows0, acc.at[dst_v.at[j0]], add=True)

            @pl.when(j0 + 2 < SCH)
            def _():
                pltpu.async_copy(hp_hbm.at[src_v.at[j0 + 2]], rows0, sem0)

            pltpu.make_async_copy(hp_hbm.at[src_v.at[j1]], rows1, sem1).wait()
            pltpu.sync_copy(rows1, acc.at[dst_v.at[j1]], add=True)
            return carry

        lax.fori_loop(0, SCH // 2, body, 0)
    plsc.subcore_barrier()
    pltpu.sync_copy(acc.at[pl.ds(s * RPZ, RPZ)], out_hbm.at[c, pl.ds(s * RPZ, RPZ)])


_RB = 400
_GRID = N // _RB

_row = pl.BlockSpec((_RB, D), lambda i: (i, 0))
_rowcol = pl.BlockSpec((_RB, 1), lambda i: (i, 0))
_wspec = pl.BlockSpec((D, D), lambda i: (0, 0))
_bspec = pl.BlockSpec((1, D), lambda i: (0, 0))
_degspec = pl.BlockSpec((NC, _RB, DW), lambda i: (0, i, 0))
_p0spec = pl.BlockSpec((1, _RB, D), lambda i: (0, i, 0))
_p1spec = pl.BlockSpec((1, _RB, D), lambda i: (1, i, 0))


def _tc1_body(degp_ref, x_ref, w_ref, hp_ref, dinv_ref):
    deg = degp_ref[0, :, :1] + degp_ref[1, :, :1] + 1.0
    dinv = lax.rsqrt(deg)
    dinv_ref[...] = dinv
    hp_ref[...] = jnp.dot(x_ref[...], w_ref[...],
                          preferred_element_type=jnp.float32) * dinv


_tc1 = pl.pallas_call(
    _tc1_body,
    grid=(_GRID,),
    in_specs=[_degspec, _row, _wspec],
    out_specs=(_row, _rowcol),
    out_shape=(jax.ShapeDtypeStruct((N, D), jnp.float32),
               jax.ShapeDtypeStruct((N, 1), jnp.float32)),
)


def _tc_mid_body(hp_ref, a0_ref, a1_ref, dinv_ref, b_ref, w_ref, out_ref):
    dinv = dinv_ref[...]
    z = dinv * (hp_ref[...] + a0_ref[0] + a1_ref[0]) + b_ref[...]
    h = jnp.maximum(z, 0.0)
    out_ref[...] = jnp.dot(h, w_ref[...],
                           preferred_element_type=jnp.float32) * dinv


_tc_mid = pl.pallas_call(
    _tc_mid_body,
    grid=(_GRID,),
    in_specs=[_row, _p0spec, _p1spec, _rowcol, _bspec, _wspec],
    out_specs=_row,
    out_shape=jax.ShapeDtypeStruct((N, D), jnp.float32),
)


def _tc_final_body(hp_ref, a0_ref, a1_ref, dinv_ref, b_ref, out_ref):
    out_ref[...] = (dinv_ref[...] * (hp_ref[...] + a0_ref[0] + a1_ref[0])
                    + b_ref[...])


_tc_final = pl.pallas_call(
    _tc_final_body,
    grid=(_GRID,),
    in_specs=[_row, _p0spec, _p1spec, _rowcol, _bspec],
    out_specs=_row,
    out_shape=jax.ShapeDtypeStruct((N, D), jnp.float32),
)


def kernel(x, edge_index, W1, b1, W2, b2, W3, b3):
    ei = edge_index.astype(jnp.int32)
    epn = E // NW
    padw = CH * K - epn
    trash = N + (jnp.arange(padw, dtype=jnp.int32) % (NP - N))
    src = jnp.concatenate(
        [ei[0].reshape(NW, epn), jnp.zeros((NW, padw), jnp.int32)], axis=1
    ).reshape(NW, CH, K)
    dst = jnp.concatenate(
        [ei[1].reshape(NW, epn), jnp.broadcast_to(trash, (NW, padw))], axis=1
    ).reshape(NW, CH, K)

    onesD = jnp.ones((K, D), jnp.float32)
    zerosD = jnp.zeros((NP, D), jnp.float32)

    degp = _deg_kernel(dst, onesD, zerosD)
    h1p, dinv = _tc1(degp, x, W1)

    b1r = b1.reshape(1, D)
    b2r = b2.reshape(1, D)
    b3r = b3.reshape(1, D)

    a1 = _agg_kernel(h1p, src, dst, zerosD)
    h2p = _tc_mid(h1p, a1, a1, dinv, b1r, W2)
    a2 = _agg_kernel(h2p, src, dst, zerosD)
    h3p = _tc_mid(h2p, a2, a2, dinv, b2r, W3)
    a3 = _agg_kernel(h3p, src, dst, zerosD)
    out = _tc_final(h3p, a3, a3, dinv, b3r)
    return out


# speedup vs baseline: 3.1100x; 2.6302x over previous
import functools

import jax
import jax.numpy as jnp
from jax import lax
from jax.experimental import pallas as pl
from jax.experimental.pallas import tpu as pltpu
from jax.experimental.pallas import tpu_sc as plsc

N = 10000
D = 128
E = 320000
NC = 2
NS = 16
NW = NC * NS
K = 128
CH = 80
SCH = 40
EPW = CH * K
EPAD = NW * EPW
NP = 10112
RPZ = NP // NS
DW = D

_MESH = plsc.VectorSubcoreMesh(core_axis_name="c", subcore_axis_name="s")


@functools.partial(
    pl.kernel,
    out_type=jax.ShapeDtypeStruct((NC, NP, D), jnp.float32),
    mesh=_MESH,
    scratch_types=[
        pltpu.VMEM((CH, K), jnp.int32),
        pltpu.VMEM((K, D), jnp.float32),
        pltpu.VMEM_SHARED((NP, D), jnp.float32),
    ],
)
def _deg_kernel(dsts_hbm, ones_hbm, zeros_hbm, out_hbm, dst_v, ones_v, dacc):
    c = lax.axis_index("c")
    s = lax.axis_index("s")
    wid = c * NS + s
    pltpu.sync_copy(zeros_hbm.at[pl.ds(s * RPZ, RPZ)], dacc.at[pl.ds(s * RPZ, RPZ)])
    pltpu.sync_copy(dsts_hbm.at[wid], dst_v)
    pltpu.sync_copy(ones_hbm, ones_v)
    plsc.subcore_barrier()

    def body(j, carry):
        pltpu.sync_copy(ones_v, dacc.at[dst_v.at[j]], add=True)
        return carry

    lax.fori_loop(0, CH, body, 0)
    plsc.subcore_barrier()
    pltpu.sync_copy(dacc.at[pl.ds(s * RPZ, RPZ)], out_hbm.at[c, pl.ds(s * RPZ, RPZ)])


@functools.partial(
    pl.kernel,
    out_type=jax.ShapeDtypeStruct((NC, NP, D), jnp.float32),
    mesh=_MESH,
    scratch_types=[
        pltpu.VMEM((SCH, K), jnp.int32),
        pltpu.VMEM((SCH, K), jnp.int32),
        pltpu.VMEM((K, D), jnp.float32),
        pltpu.VMEM((K, D), jnp.float32),
        pltpu.VMEM_SHARED((NP, D), jnp.float32),
        pltpu.SemaphoreType.DMA,
        pltpu.SemaphoreType.DMA,
    ],
)
def _agg_kernel(hp_hbm, srcs_hbm, dsts_hbm, zeros_hbm, out_hbm,
                src_v, dst_v, rows0, rows1, acc, sem0, sem1):
    c = lax.axis_index("c")
    s = lax.axis_index("s")
    wid = c * NS + s
    pltpu.sync_copy(zeros_hbm.at[pl.ds(s * RPZ, RPZ)], acc.at[pl.ds(s * RPZ, RPZ)])

    for h in range(CH // SCH):
        pltpu.sync_copy(srcs_hbm.at[wid, pl.ds(h * SCH, SCH)], src_v)
        pltpu.sync_copy(dsts_hbm.at[wid, pl.ds(h * SCH, SCH)], dst_v)
        if h == 0:
            plsc.subcore_barrier()
        pltpu.async_copy(hp_hbm.at[src_v.at[0]], rows0, sem0)

        def body(i, carry):
            j0 = 2 * i
            j1 = j0 + 1
            pltpu.async_copy(hp_hbm.at[src_v.at[j1]], rows1, sem1)
            pltpu.make_async_copy(hp_hbm.at[src_v.at[j0]], rows0, sem0).wait()
            pltpu.sync_copy(rows0, acc.at[dst_v.at[j0]], add=True)

            @pl.when(j0 + 2 < SCH)
            def _():
                pltpu.async_copy(hp_hbm.at[src_v.at[j0 + 2]], rows0, sem0)

            pltpu.make_async_copy(hp_hbm.at[src_v.at[j1]], rows1, sem1).wait()
            pltpu.sync_copy(rows1, acc.at[dst_v.at[j1]], add=True)
            return carry

        lax.fori_loop(0, SCH // 2, body, 0)
    plsc.subcore_barrier()
    pltpu.sync_copy(acc.at[pl.ds(s * RPZ, RPZ)], out_hbm.at[c, pl.ds(s * RPZ, RPZ)])


_RB = 400
_GRID = N // _RB

_row = pl.BlockSpec((_RB, D), lambda i: (i, 0))
_rowcol = pl.BlockSpec((_RB, 1), lambda i: (i, 0))
_wspec = pl.BlockSpec((D, D), lambda i: (0, 0))
_bspec = pl.BlockSpec((1, D), lambda i: (0, 0))
_degspec = pl.BlockSpec((NC, _RB, DW), lambda i: (0, i, 0))
_p0spec = pl.BlockSpec((1, _RB, D), lambda i: (0, i, 0))
_p1spec = pl.BlockSpec((1, _RB, D), lambda i: (1, i, 0))


def _tc1_body(degp_ref, x_ref, w_ref, hp_ref, dinv_ref):
    deg = degp_ref[0, :, :1] + degp_ref[1, :, :1] + 1.0
    dinv = lax.rsqrt(deg)
    dinv_ref[...] = dinv
    hp_ref[...] = jnp.dot(x_ref[...], w_ref[...],
                          preferred_element_type=jnp.float32) * dinv


_tc1 = pl.pallas_call(
    _tc1_body,
    grid=(_GRID,),
    in_specs=[_degspec, _row, _wspec],
    out_specs=(_row, _rowcol),
    out_shape=(jax.ShapeDtypeStruct((N, D), jnp.float32),
               jax.ShapeDtypeStruct((N, 1), jnp.float32)),
)


def _tc_mid_body(hp_ref, a0_ref, a1_ref, dinv_ref, b_ref, w_ref, out_ref):
    dinv = dinv_ref[...]
    z = dinv * (hp_ref[...] + a0_ref[0] + a1_ref[0]) + b_ref[...]
    h = jnp.maximum(z, 0.0)
    out_ref[...] = jnp.dot(h, w_ref[...],
                           preferred_element_type=jnp.float32) * dinv


_tc_mid = pl.pallas_call(
    _tc_mid_body,
    grid=(_GRID,),
    in_specs=[_row, _p0spec, _p1spec, _rowcol, _bspec, _wspec],
    out_specs=_row,
    out_shape=jax.ShapeDtypeStruct((N, D), jnp.float32),
)


def _tc_final_body(hp_ref, a0_ref, a1_ref, dinv_ref, b_ref, out_ref):
    out_ref[...] = (dinv_ref[...] * (hp_ref[...] + a0_ref[0] + a1_ref[0])
                    + b_ref[...])


_tc_final = pl.pallas_call(
    _tc_final_body,
    grid=(_GRID,),
    in_specs=[_row, _p0spec, _p1spec, _rowcol, _bspec],
    out_specs=_row,
    out_shape=jax.ShapeDtypeStruct((N, D), jnp.float32),
)


def kernel(x, edge_index, W1, b1, W2, b2, W3, b3):
    ei = edge_index.astype(jnp.int32)
    epn = E // NW
    padw = CH * K - epn
    sub = (jnp.arange(NW, dtype=jnp.int32) % NS)[:, None]
    trash = N + sub * 7 + (jnp.arange(padw, dtype=jnp.int32)[None, :] % 7)
    padsrc = jnp.broadcast_to(
        (jnp.arange(padw, dtype=jnp.int32) * 41) % N, (NW, padw))
    src = jnp.concatenate(
        [ei[0].reshape(NW, epn), padsrc], axis=1).reshape(NW, CH, K)
    dst = jnp.concatenate(
        [ei[1].reshape(NW, epn), trash], axis=1).reshape(NW, CH, K)

    onesD = jnp.ones((K, D), jnp.float32)
    zerosD = jnp.zeros((NP, D), jnp.float32)

    degp = _deg_kernel(dst, onesD, zerosD)
    h1p, dinv = _tc1(degp, x, W1)

    b1r = b1.reshape(1, D)
    b2r = b2.reshape(1, D)
    b3r = b3.reshape(1, D)

    a1 = _agg_kernel(h1p, src, dst, zerosD)
    h2p = _tc_mid(h1p, a1, a1, dinv, b1r, W2)
    a2 = _agg_kernel(h2p, src, dst, zerosD)
    h3p = _tc_mid(h2p, a2, a2, dinv, b2r, W3)
    a3 = _agg_kernel(h3p, src, dst, zerosD)
    out = _tc_final(h3p, a3, a3, dinv, b3r)
    return out


# speedup vs baseline: 3.3166x; 1.0664x over previous
import functools

import jax
import jax.numpy as jnp
from jax import lax
from jax.experimental import pallas as pl
from jax.experimental.pallas import tpu as pltpu
from jax.experimental.pallas import tpu_sc as plsc

N = 10000
D = 128
E = 320000
NC = 2
NS = 16
NW = NC * NS
K = 128
CH = 80
SCH = 40
EPW = CH * K
EPAD = NW * EPW
NP = 10112
RPZ = NP // NS
DW = D

_MESH = plsc.VectorSubcoreMesh(core_axis_name="c", subcore_axis_name="s")


@functools.partial(
    pl.kernel,
    out_type=jax.ShapeDtypeStruct((NC, NP, D), jnp.float32),
    mesh=_MESH,
    scratch_types=[
        pltpu.VMEM((CH, K), jnp.int32),
        pltpu.VMEM((K, D), jnp.float32),
        pltpu.VMEM_SHARED((NP, D), jnp.float32),
    ],
)
def _deg_kernel(dsts_hbm, ones_hbm, zeros_hbm, out_hbm, dst_v, ones_v, dacc):
    c = lax.axis_index("c")
    s = lax.axis_index("s")
    wid = c * NS + s
    pltpu.sync_copy(zeros_hbm.at[pl.ds(s * RPZ, RPZ)], dacc.at[pl.ds(s * RPZ, RPZ)])
    pltpu.sync_copy(dsts_hbm.at[wid], dst_v)
    pltpu.sync_copy(ones_hbm, ones_v)
    plsc.subcore_barrier()

    def body(j, carry):
        pltpu.sync_copy(ones_v, dacc.at[dst_v.at[j]], add=True)
        return carry

    lax.fori_loop(0, CH, body, 0)
    plsc.subcore_barrier()
    pltpu.sync_copy(dacc.at[pl.ds(s * RPZ, RPZ)], out_hbm.at[c, pl.ds(s * RPZ, RPZ)])


@functools.partial(
    pl.kernel,
    out_type=jax.ShapeDtypeStruct((NC, NP, D), jnp.float32),
    mesh=_MESH,
    scratch_types=[
        pltpu.VMEM((SCH, K), jnp.int32),
        pltpu.VMEM((SCH, K), jnp.int32),
        pltpu.VMEM((K, D), jnp.float32),
        pltpu.VMEM((K, D), jnp.float32),
        pltpu.VMEM_SHARED((NP, D), jnp.float32),
        pltpu.SemaphoreType.DMA,
        pltpu.SemaphoreType.DMA,
    ],
)
def _agg_kernel(hp_hbm, srcs_hbm, dsts_hbm, zeros_hbm, out_hbm,
                src_v, dst_v, rows0, rows1, acc, sem0, sem1):
    c = lax.axis_index("c")
    s = lax.axis_index("s")
    wid = c * NS + s
    pltpu.sync_copy(zeros_hbm.at[pl.ds(s * RPZ, RPZ)], acc.at[pl.ds(s * RPZ, RPZ)])

    for h in range(CH // SCH):
        pltpu.sync_copy(srcs_hbm.at[wid, pl.ds(h * SCH, SCH)], src_v)
        pltpu.sync_copy(dsts_hbm.at[wid, pl.ds(h * SCH, SCH)], dst_v)
        if h == 0:
            plsc.subcore_barrier()
        pltpu.async_copy(hp_hbm.at[src_v.at[0]], rows0, sem0)

        def body(i, carry):
            j0 = 2 * i
            j1 = j0 + 1
            pltpu.async_copy(hp_hbm.at[src_v.at[j1]], rows1, sem1)
            pltpu.make_async_copy(hp_hbm.at[src_v.at[j0]], rows0, sem0).wait()
            pltpu.sync_copy(rows0, acc.at[dst_v.at[j0]], add=True)

            @pl.when(j0 + 2 < SCH)
            def _():
                pltpu.async_copy(hp_hbm.at[src_v.at[j0 + 2]], rows0, sem0)

            pltpu.make_async_copy(hp_hbm.at[src_v.at[j1]], rows1, sem1).wait()
            pltpu.sync_copy(rows1, acc.at[dst_v.at[j1]], add=True)
            return carry

        lax.fori_loop(0, SCH // 2, body, 0)
    plsc.subcore_barrier()
    pltpu.sync_copy(acc.at[pl.ds(s * RPZ, RPZ)], out_hbm.at[c, pl.ds(s * RPZ, RPZ)])


_RB = 1000
_GRID = N // _RB

_row = pl.BlockSpec((_RB, D), lambda i: (i, 0))
_rowcol = pl.BlockSpec((_RB, 1), lambda i: (i, 0))
_wspec = pl.BlockSpec((D, D), lambda i: (0, 0))
_bspec = pl.BlockSpec((1, D), lambda i: (0, 0))
_degspec = pl.BlockSpec((NC, _RB, DW), lambda i: (0, i, 0))
_p0spec = pl.BlockSpec((1, _RB, D), lambda i: (0, i, 0))
_p1spec = pl.BlockSpec((1, _RB, D), lambda i: (1, i, 0))


def _tc_mm_body(x_ref, w_ref, out_ref):
    out_ref[...] = jnp.dot(x_ref[...], w_ref[...],
                           preferred_element_type=jnp.float32)


_tc_mm = pl.pallas_call(
    _tc_mm_body,
    grid=(_GRID,),
    in_specs=[_row, _wspec],
    out_specs=_row,
    out_shape=jax.ShapeDtypeStruct((N, D), jnp.float32),
)


def _tc_scale_body(degp_ref, g_ref, hp_ref, dinv_ref):
    deg = degp_ref[0, :, :1] + degp_ref[1, :, :1] + 1.0
    dinv = lax.rsqrt(deg)
    dinv_ref[...] = dinv
    hp_ref[...] = g_ref[...] * dinv


_tc_scale = pl.pallas_call(
    _tc_scale_body,
    grid=(_GRID,),
    in_specs=[_degspec, _row],
    out_specs=(_row, _rowcol),
    out_shape=(jax.ShapeDtypeStruct((N, D), jnp.float32),
               jax.ShapeDtypeStruct((N, 1), jnp.float32)),
)


def _tc_mid_body(hp_ref, a0_ref, a1_ref, dinv_ref, b_ref, w_ref, out_ref):
    dinv = dinv_ref[...]
    z = dinv * (hp_ref[...] + a0_ref[0] + a1_ref[0]) + b_ref[...]
    h = jnp.maximum(z, 0.0)
    out_ref[...] = jnp.dot(h, w_ref[...],
                           preferred_element_type=jnp.float32) * dinv


_tc_mid = pl.pallas_call(
    _tc_mid_body,
    grid=(_GRID,),
    in_specs=[_row, _p0spec, _p1spec, _rowcol, _bspec, _wspec],
    out_specs=_row,
    out_shape=jax.ShapeDtypeStruct((N, D), jnp.float32),
)


def _tc_final_body(hp_ref, a0_ref, a1_ref, dinv_ref, b_ref, out_ref):
    out_ref[...] = (dinv_ref[...] * (hp_ref[...] + a0_ref[0] + a1_ref[0])
                    + b_ref[...])


_tc_final = pl.pallas_call(
    _tc_final_body,
    grid=(_GRID,),
    in_specs=[_row, _p0spec, _p1spec, _rowcol, _bspec],
    out_specs=_row,
    out_shape=jax.ShapeDtypeStruct((N, D), jnp.float32),
)


def kernel(x, edge_index, W1, b1, W2, b2, W3, b3):
    ei = edge_index.astype(jnp.int32)
    epn = E // NW
    padw = CH * K - epn
    sub = (jnp.arange(NW, dtype=jnp.int32) % NS)[:, None]
    trash = N + sub * 7 + (jnp.arange(padw, dtype=jnp.int32)[None, :] % 7)
    padsrc = jnp.broadcast_to(
        (jnp.arange(padw, dtype=jnp.int32) * 41) % N, (NW, padw))
    src = jnp.concatenate(
        [ei[0].reshape(NW, epn), padsrc], axis=1).reshape(NW, CH, K)
    dst = jnp.concatenate(
        [ei[1].reshape(NW, epn), trash], axis=1).reshape(NW, CH, K)

    onesD = jnp.ones((K, D), jnp.float32)
    zerosD = jnp.zeros((NP, D), jnp.float32)

    g1 = _tc_mm(x, W1)
    degp = _deg_kernel(dst, onesD, zerosD)
    h1p, dinv = _tc_scale(degp, g1)

    b1r = b1.reshape(1, D)
    b2r = b2.reshape(1, D)
    b3r = b3.reshape(1, D)

    a1 = _agg_kernel(h1p, src, dst, zerosD)
    h2p = _tc_mid(h1p, a1, a1, dinv, b1r, W2)
    a2 = _agg_kernel(h2p, src, dst, zerosD)
    h3p = _tc_mid(h2p, a2, a2, dinv, b2r, W3)
    a3 = _agg_kernel(h3p, src, dst, zerosD)
    out = _tc_final(h3p, a3, a3, dinv, b3r)
    return out


# speedup vs baseline: 3.4122x; 1.0288x over previous
import functools

import jax
import jax.numpy as jnp
from jax import lax
from jax.experimental import pallas as pl
from jax.experimental.pallas import tpu as pltpu
from jax.experimental.pallas import tpu_sc as plsc

N = 10000
D = 128
E = 320000
NC = 2
NS = 16
NW = NC * NS
K = 128
CH = 80
SCH = 40
EPW = CH * K
EPAD = NW * EPW
NP = 10112
RPZ = NP // NS
DW = D

_MESH = plsc.VectorSubcoreMesh(core_axis_name="c", subcore_axis_name="s")


@functools.partial(
    pl.kernel,
    out_type=jax.ShapeDtypeStruct((NC, NP, D), jnp.float32),
    mesh=_MESH,
    scratch_types=[
        pltpu.VMEM((CH, K), jnp.int32),
        pltpu.VMEM((K, D), jnp.float32),
        pltpu.VMEM_SHARED((NP, D), jnp.float32),
    ],
)
def _deg_kernel(dsts_hbm, ones_hbm, zeros_hbm, out_hbm, dst_v, ones_v, dacc):
    c = lax.axis_index("c")
    s = lax.axis_index("s")
    wid = c * NS + s
    pltpu.sync_copy(zeros_hbm.at[pl.ds(s * RPZ, RPZ)], dacc.at[pl.ds(s * RPZ, RPZ)])
    pltpu.sync_copy(dsts_hbm.at[wid], dst_v)
    pltpu.sync_copy(ones_hbm, ones_v)
    plsc.subcore_barrier()

    def body(j, carry):
        pltpu.sync_copy(ones_v, dacc.at[dst_v.at[j]], add=True)
        return carry

    lax.fori_loop(0, CH, body, 0)
    plsc.subcore_barrier()
    pltpu.sync_copy(dacc.at[pl.ds(s * RPZ, RPZ)], out_hbm.at[c, pl.ds(s * RPZ, RPZ)])


@functools.partial(
    pl.kernel,
    out_type=jax.ShapeDtypeStruct((NC, NP, D), jnp.float32),
    mesh=_MESH,
    scratch_types=[
        pltpu.VMEM((SCH, K), jnp.int32),
        pltpu.VMEM((SCH, K), jnp.int32),
        pltpu.VMEM((K, D), jnp.float32),
        pltpu.VMEM((K, D), jnp.float32),
        pltpu.VMEM_SHARED((NP, D), jnp.float32),
        pltpu.SemaphoreType.DMA,
        pltpu.SemaphoreType.DMA,
    ],
)
def _agg_kernel(hp_hbm, srcs_hbm, dsts_hbm, zeros_hbm, out_hbm,
                src_v, dst_v, rows0, rows1, acc, sem0, sem1):
    c = lax.axis_index("c")
    s = lax.axis_index("s")
    wid = c * NS + s

    zv = jnp.zeros((16,), jnp.float32)

    def zrow(r, carry):
        for g in range(8):
            rows0[r, pl.ds(g * 16, 16)] = zv
        return carry

    lax.fori_loop(0, K, zrow, 0)
    base = s * RPZ
    for t in range(4):
        pltpu.sync_copy(rows0, acc.at[pl.ds(base + t * K, K)])
    pltpu.sync_copy(rows0.at[pl.ds(0, RPZ - 4 * K)],
                    acc.at[pl.ds(base + 4 * K, RPZ - 4 * K)])

    for h in range(CH // SCH):
        pltpu.sync_copy(srcs_hbm.at[wid, pl.ds(h * SCH, SCH)], src_v)
        pltpu.sync_copy(dsts_hbm.at[wid, pl.ds(h * SCH, SCH)], dst_v)
        if h == 0:
            plsc.subcore_barrier()
        pltpu.async_copy(hp_hbm.at[src_v.at[0]], rows0, sem0)

        def body(i, carry):
            j0 = 2 * i
            j1 = j0 + 1
            pltpu.async_copy(hp_hbm.at[src_v.at[j1]], rows1, sem1)
            pltpu.make_async_copy(hp_hbm.at[src_v.at[j0]], rows0, sem0).wait()
            pltpu.sync_copy(rows0, acc.at[dst_v.at[j0]], add=True)

            @pl.when(j0 + 2 < SCH)
            def _():
                pltpu.async_copy(hp_hbm.at[src_v.at[j0 + 2]], rows0, sem0)

            pltpu.make_async_copy(hp_hbm.at[src_v.at[j1]], rows1, sem1).wait()
            pltpu.sync_copy(rows1, acc.at[dst_v.at[j1]], add=True)
            return carry

        lax.fori_loop(0, SCH // 2, body, 0)
    plsc.subcore_barrier()
    pltpu.sync_copy(acc.at[pl.ds(s * RPZ, RPZ)], out_hbm.at[c, pl.ds(s * RPZ, RPZ)])


_RB = 2000
_GRID = N // _RB

_row = pl.BlockSpec((_RB, D), lambda i: (i, 0))
_rowcol = pl.BlockSpec((_RB, 1), lambda i: (i, 0))
_wspec = pl.BlockSpec((D, D), lambda i: (0, 0))
_bspec = pl.BlockSpec((1, D), lambda i: (0, 0))
_degspec = pl.BlockSpec((NC, _RB, DW), lambda i: (0, i, 0))
_p0spec = pl.BlockSpec((1, _RB, D), lambda i: (0, i, 0))
_p1spec = pl.BlockSpec((1, _RB, D), lambda i: (1, i, 0))


def _tc_mm_body(x_ref, w_ref, out_ref):
    out_ref[...] = jnp.dot(x_ref[...], w_ref[...],
                           preferred_element_type=jnp.float32)


_tc_mm = pl.pallas_call(
    _tc_mm_body,
    grid=(_GRID,),
    in_specs=[_row, _wspec],
    out_specs=_row,
    out_shape=jax.ShapeDtypeStruct((N, D), jnp.float32),
)


def _tc_scale_body(degp_ref, g_ref, hp_ref, dinv_ref):
    deg = degp_ref[0, :, :1] + degp_ref[1, :, :1] + 1.0
    dinv = lax.rsqrt(deg)
    dinv_ref[...] = dinv
    hp_ref[...] = g_ref[...] * dinv


_tc_scale = pl.pallas_call(
    _tc_scale_body,
    grid=(_GRID,),
    in_specs=[_degspec, _row],
    out_specs=(_row, _rowcol),
    out_shape=(jax.ShapeDtypeStruct((N, D), jnp.float32),
               jax.ShapeDtypeStruct((N, 1), jnp.float32)),
)


def _tc_mid_body(hp_ref, a0_ref, a1_ref, dinv_ref, b_ref, w_ref, out_ref):
    dinv = dinv_ref[...]
    z = dinv * (hp_ref[...] + a0_ref[0] + a1_ref[0]) + b_ref[...]
    h = jnp.maximum(z, 0.0)
    out_ref[...] = jnp.dot(h, w_ref[...],
                           preferred_element_type=jnp.float32) * dinv


_tc_mid = pl.pallas_call(
    _tc_mid_body,
    grid=(_GRID,),
    in_specs=[_row, _p0spec, _p1spec, _rowcol, _bspec, _wspec],
    out_specs=_row,
    out_shape=jax.ShapeDtypeStruct((N, D), jnp.float32),
)


def _tc_final_body(hp_ref, a0_ref, a1_ref, dinv_ref, b_ref, out_ref):
    out_ref[...] = (dinv_ref[...] * (hp_ref[...] + a0_ref[0] + a1_ref[0])
                    + b_ref[...])


_tc_final = pl.pallas_call(
    _tc_final_body,
    grid=(_GRID,),
    in_specs=[_row, _p0spec, _p1spec, _rowcol, _bspec],
    out_specs=_row,
    out_shape=jax.ShapeDtypeStruct((N, D), jnp.float32),
)


def kernel(x, edge_index, W1, b1, W2, b2, W3, b3):
    ei = edge_index.astype(jnp.int32)
    epn = E // NW
    padw = CH * K - epn
    sub = (jnp.arange(NW, dtype=jnp.int32) % NS)[:, None]
    trash = N + sub * 7 + (jnp.arange(padw, dtype=jnp.int32)[None, :] % 7)
    padsrc = jnp.broadcast_to(
        (jnp.arange(padw, dtype=jnp.int32) * 41) % N, (NW, padw))
    src = jnp.concatenate(
        [ei[0].reshape(NW, epn), padsrc], axis=1).reshape(NW, CH, K)
    dst = jnp.concatenate(
        [ei[1].reshape(NW, epn), trash], axis=1).reshape(NW, CH, K)

    onesD = jnp.ones((K, D), jnp.float32)
    zerosD = jnp.zeros((NP, D), jnp.float32)

    g1 = _tc_mm(x, W1)
    degp = _deg_kernel(dst, onesD, zerosD)
    h1p, dinv = _tc_scale(degp, g1)

    b1r = b1.reshape(1, D)
    b2r = b2.reshape(1, D)
    b3r = b3.reshape(1, D)

    a1 = _agg_kernel(h1p, src, dst, zerosD)
    h2p = _tc_mid(h1p, a1, a1, dinv, b1r, W2)
    a2 = _agg_kernel(h2p, src, dst, zerosD)
    h3p = _tc_mid(h2p, a2, a2, dinv, b2r, W3)
    a3 = _agg_kernel(h3p, src, dst, zerosD)
    out = _tc_final(h3p, a3, a3, dinv, b3r)
    return out
